# Initial kernel scaffold; baseline (speedup 1.0000x reference)
import jax
import jax.numpy as jnp
from jax.experimental import pallas as pl


def kernel(feat, edge_index, query, src_key_weight, dst_key_weight, src_key_bias, dst_key_bias, edge_key_weight, edge_key_bias, src_value_weight, dst_value_weight, src_value_bias, dst_value_bias, edge_value_weight, edge_value_bias, key_norm_gamma, key_norm_beta, value_norm_gamma, value_norm_beta, out_norm_gamma, out_norm_beta, i):
    raise NotImplementedError("write your pallas kernel here")


# speedup vs baseline: 1.9682x; 1.9682x over previous
import functools

import jax
import jax.numpy as jnp
from jax import lax
from jax.experimental import pallas as pl
from jax.experimental.pallas import tpu as pltpu
import jax.experimental.pallas.tpu_sc as plsc

N = 5000
E = 10000
D = 64
H = 4
DH = 16
FF = 8
HF = H * FF
LW = H * FF * D
LW2 = H * FF * DH

NW = 32
EW = 320
EP = NW * EW
NP = 5120
ROW = 80
BE = 80
BN = 128
EPS = 1e-5

@functools.cache
def _mesh():
    return plsc.VectorSubcoreMesh(core_axis_name="c", subcore_axis_name="s")


def _worker_id():
    return lax.axis_index("s") * 2 + lax.axis_index("c")


def _gather_body(feat_hbm, src_hbm, dst_hbm, fs_hbm, fd_hbm,
                 sidx, didx, srows, drows, sem):
    base = _worker_id() * EW
    pltpu.sync_copy(src_hbm.at[pl.ds(base, EW)], sidx)
    pltpu.sync_copy(dst_hbm.at[pl.ds(base, EW)], didx)
    pltpu.async_copy(feat_hbm.at[sidx], srows, sem).wait()
    pltpu.async_copy(feat_hbm.at[didx], drows, sem).wait()
    pltpu.sync_copy(srows, fs_hbm.at[pl.ds(base, EW)])
    pltpu.sync_copy(drows, fd_hbm.at[pl.ds(base, EW)])


@jax.jit
def _stage_g(feat, src_p, dst_p):
    return pl.kernel(
        _gather_body,
        out_type=[jax.ShapeDtypeStruct((EP, D), jnp.float32),
                  jax.ShapeDtypeStruct((EP, D), jnp.float32)],
        mesh=_mesh(),
        scratch_types=[
            pltpu.VMEM((EW,), jnp.int32),
            pltpu.VMEM((EW,), jnp.int32),
            pltpu.VMEM((EW, D), jnp.float32),
            pltpu.VMEM((EW, D), jnp.float32),
            pltpu.SemaphoreType.DMA,
        ],
        compiler_params=pltpu.CompilerParams(use_tc_tiling_on_sc=False, needs_layout_passes=False),
    )(feat, src_p, dst_p)


def _ln16(x):
    m = jnp.mean(x, axis=-1, keepdims=True)
    c = x - m
    v = jnp.mean(c * c, axis=-1, keepdims=True)
    return c * lax.rsqrt(v + EPS)


def _ffn_path(fst, fdt, uw, vw, ub, vb, ew, eb, r1, rex, r2):
    p = uw * fst + vw * fdt
    h1 = jnp.dot(p, r1, preferred_element_type=jnp.float32) + ub + vb
    h1 = jnp.maximum(h1, 0.0)
    h1e = jnp.dot(h1, rex, preferred_element_type=jnp.float32)
    acc = jnp.dot(ew * h1e, r2, preferred_element_type=jnp.float32) + eb
    outs = [_ln16(acc[:, h * DH:(h + 1) * DH]) for h in range(H)]
    return jnp.concatenate(outs, axis=1)


def _ffn_body(fs, fd, skw, dkw, skb, dkb, ekw, ekb,
              svw, dvw, svb, dvb, evw, evb, q,
              rt, r1, rex, r2, r3,
              logits_ref, value_ref, m_ref):
    fst = jnp.dot(fs[...], rt[...], preferred_element_type=jnp.float32)
    fdt = jnp.dot(fd[...], rt[...], preferred_element_type=jnp.float32)
    key = _ffn_path(fst, fdt, skw[...], dkw[...], skb[...], dkb[...],
                    ekw[...], ekb[...], r1[...], rex[...], r2[...])
    logits = jnp.dot(key * q[...], r3[...],
                     preferred_element_type=jnp.float32)
    logits_ref[...] = logits
    val = _ffn_path(fst, fdt, svw[...], dvw[...], svb[...], dvb[...],
                    evw[...], evb[...], r1[...], rex[...], r2[...])
    value_ref[...] = val
    bmax = jnp.max(logits, axis=0, keepdims=True)
    @pl.when(pl.program_id(0) == 0)
    def _():
        m_ref[...] = bmax
    @pl.when(pl.program_id(0) > 0)
    def _():
        m_ref[...] = jnp.maximum(m_ref[...], bmax)


@jax.jit
def _stage_a(fs, fd, skw, dkw, skb, dkb, ekw, ekb,
             svw, dvw, svb, dvb, evw, evb, q):
    ar = jnp.arange
    rt = (ar(D)[:, None] == ar(LW)[None, :] % D).astype(jnp.float32)
    r1 = (ar(LW)[:, None] // D == ar(HF)[None, :]).astype(jnp.float32)
    rex = (ar(HF)[:, None] == ar(LW2)[None, :] // DH).astype(jnp.float32)
    r2 = ((ar(LW2)[:, None] // (FF * DH) == ar(D)[None, :] // DH)
          & (ar(LW2)[:, None] % DH == ar(D)[None, :] % DH)).astype(jnp.float32)
    r3 = (ar(D)[:, None] // DH == ar(H)[None, :]).astype(jnp.float32)

    grid = E // BE
    eb_spec = lambda w: pl.BlockSpec((BE, w), lambda i: (i, 0))
    cb_spec = lambda a, b: pl.BlockSpec((a, b), lambda i: (0, 0))
    return pl.pallas_call(
        _ffn_body,
        grid=(grid,),
        in_specs=[
            eb_spec(D), eb_spec(D),
            eb_spec(LW), eb_spec(LW), eb_spec(HF), eb_spec(HF),
            eb_spec(LW2), eb_spec(D),
            eb_spec(LW), eb_spec(LW), eb_spec(HF), eb_spec(HF),
            eb_spec(LW2), eb_spec(D), eb_spec(D),
            cb_spec(D, LW), cb_spec(LW, HF), cb_spec(HF, LW2),
            cb_spec(LW2, D), cb_spec(D, H),
        ],
        out_specs=[
            pl.BlockSpec((BE, H), lambda i: (i, 0)),
            pl.BlockSpec((BE, D), lambda i: (i, 0)),
            pl.BlockSpec((1, H), lambda i: (0, 0)),
        ],
        out_shape=[
            jax.ShapeDtypeStruct((E, H), jnp.float32),
            jax.ShapeDtypeStruct((E, D), jnp.float32),
            jax.ShapeDtypeStruct((1, H), jnp.float32),
        ],
    )(fs, fd, skw, dkw, skb, dkb, ekw, ekb,
      svw, dvw, svb, dvb, evw, evb, q, rt, r1, rex, r2, r3)


def _scatter_body(lg_hbm, val_hbm, dst_hbm, m16_hbm,
                  parts_hbm, z_hbm,
                  lgv, zv, dstv, valv, rowbuf, m16v, table):
    c = lax.axis_index("c")
    s = lax.axis_index("s")
    w = s * 2 + c
    base = w * EW
    nrows = NP // 16

    def zrow(idx, _):
        r = idx // (ROW // 16)
        k = idx % (ROW // 16)
        rowbuf[r, pl.ds(k * 16, 16)] = jnp.zeros((16,), jnp.float32)
        return 0
    lax.fori_loop(0, EW * (ROW // 16), zrow, 0)
    pltpu.sync_copy(rowbuf, table.at[pl.ds(s * nrows, nrows)])
    plsc.subcore_barrier()

    pltpu.sync_copy(lg_hbm.at[pl.ds(base * H, EW * H)], lgv.at[pl.ds(0, EW * H)])
    pltpu.sync_copy(dst_hbm.at[pl.ds(base, EW)], dstv)
    pltpu.sync_copy(val_hbm.at[pl.ds(base, EW)], valv)
    pltpu.sync_copy(m16_hbm, m16v)

    mv = m16v[...]

    def zcomp(j, _):
        zj = jnp.exp(lgv[pl.ds(j * 16, 16)] - mv)
        zv[pl.ds(j * 16, 16)] = zj
        return 0
    lax.fori_loop(0, EW * H // 16, zcomp, 0)

    lanes = lax.iota(jnp.int32, 16)
    lane4 = lanes & 3

    def fill(l, _):
        for h in range(H):
            zs = plsc.load_gather(
                zv, [jnp.full((16,), l * H + h, jnp.int32)])
            rowbuf[l, pl.ds(h * DH, DH)] = valv[l, pl.ds(h * DH, DH)] * zs
        zg = plsc.load_gather(zv, [l * H + lane4])
        rowbuf[l, pl.ds(D, 16)] = jnp.where(lanes < H, zg, 0.0)
        return 0
    lax.fori_loop(0, EW, fill, 0)

    pltpu.sync_copy(rowbuf, table.at[dstv], add=True)
    plsc.subcore_barrier()

    pltpu.sync_copy(table.at[pl.ds(s * nrows, nrows)],
                    parts_hbm.at[c, pl.ds(s * nrows, nrows)])
    pltpu.sync_copy(zv.at[pl.ds(0, EW * H)], z_hbm.at[pl.ds(base * H, EW * H)])


@jax.jit
def _stage_b(lg_flat, val_p, dst_p, m16):
    return pl.kernel(
        _scatter_body,
        out_type=[jax.ShapeDtypeStruct((2, NP, ROW), jnp.float32),
                  jax.ShapeDtypeStruct((EP * H,), jnp.float32)],
        mesh=_mesh(),
        scratch_types=[
            pltpu.VMEM((EW * H,), jnp.float32),
            pltpu.VMEM((EW * H + 16,), jnp.float32),
            pltpu.VMEM((EW,), jnp.int32),
            pltpu.VMEM((EW, D), jnp.float32),
            pltpu.VMEM((EW, ROW), jnp.float32),
            pltpu.VMEM((16,), jnp.float32),
            pltpu.VMEM_SHARED((NP, ROW), jnp.float32),
        ],
        compiler_params=pltpu.CompilerParams(use_tc_tiling_on_sc=False, needs_layout_passes=False),
    )(lg_flat, val_p, dst_p, m16)


def _combine_body(parts_ref, agg_ref, ssum_ref):
    p = parts_ref[0] + parts_ref[1]
    ss = p[:, D:D + H]
    ss_safe = jnp.where(ss == 0.0, 1.0, ss)
    for h in range(H):
        x = p[:, h * DH:(h + 1) * DH] / ss_safe[:, h:h + 1]
        agg_ref[:, h * DH:(h + 1) * DH] = _ln16(x)
    ssum_ref[...] = ss


@jax.jit
def _stage_d(parts):
    return pl.pallas_call(
        _combine_body,
        grid=(NP // BN,),
        in_specs=[pl.BlockSpec((2, BN, ROW), lambda i: (0, i, 0))],
        out_specs=[pl.BlockSpec((BN, D), lambda i: (i, 0)),
                   pl.BlockSpec((BN, H), lambda i: (i, 0))],
        out_shape=[jax.ShapeDtypeStruct((NP, D), jnp.float32),
                   jax.ShapeDtypeStruct((NP, H), jnp.float32)],
    )(parts)


def _attn_body(z_hbm, dst_hbm, ss_hbm, attn_hbm, ssv, zv, dstv, attnv):
    base = _worker_id() * EW
    pltpu.sync_copy(ss_hbm, ssv)
    pltpu.sync_copy(z_hbm.at[pl.ds(base * H, EW * H)], zv)
    pltpu.sync_copy(dst_hbm.at[pl.ds(base, EW)], dstv)

    lanes = lax.iota(jnp.int32, 16)
    lsh = lanes >> 2
    lane4 = lanes & 3

    def step(j, _):
        dg = plsc.load_gather(dstv, [j * 4 + lsh])
        sv = plsc.load_gather(ssv, [dg * H + lane4])
        attnv[pl.ds(j * 16, 16)] = zv[pl.ds(j * 16, 16)] / sv
        return 0
    lax.fori_loop(0, EW * H // 16, step, 0)
    pltpu.sync_copy(attnv, attn_hbm.at[pl.ds(base * H, EW * H)])


@jax.jit
def _stage_c2(z_flat, dst_p, ss_flat):
    return pl.kernel(
        _attn_body,
        out_type=jax.ShapeDtypeStruct((EP * H,), jnp.float32),
        mesh=_mesh(),
        scratch_types=[
            pltpu.VMEM((NP * H,), jnp.float32),
            pltpu.VMEM((EW * H,), jnp.float32),
            pltpu.VMEM((EW,), jnp.int32),
            pltpu.VMEM((EW * H,), jnp.float32),
        ],
        compiler_params=pltpu.CompilerParams(use_tc_tiling_on_sc=False, needs_layout_passes=False),
    )(z_flat, dst_p, ss_flat)


def kernel(feat, edge_index, query, src_key_weight, dst_key_weight,
           src_key_bias, dst_key_bias, edge_key_weight, edge_key_bias,
           src_value_weight, dst_value_weight, src_value_bias, dst_value_bias,
           edge_value_weight, edge_value_bias, key_norm_gamma, key_norm_beta,
           value_norm_gamma, value_norm_beta, out_norm_gamma, out_norm_beta, i):
    f32 = jnp.float32
    ei = edge_index.astype(jnp.int32)
    src = ei[0]
    dst = ei[1]
    src_g = jnp.pad(src, (0, EP - E))
    dst_g = jnp.pad(dst, (0, EP - E))
    dst_b = jnp.pad(dst, (0, EP - E), constant_values=N)

    sq = lambda w, s: w.reshape(E, s).astype(f32)
    skw = sq(src_key_weight, LW)
    dkw = sq(dst_key_weight, LW)
    skb = sq(src_key_bias, HF)
    dkb = sq(dst_key_bias, HF)
    ekw = sq(edge_key_weight, LW2)
    ekb = sq(edge_key_bias, D)
    svw = sq(src_value_weight, LW)
    dvw = sq(dst_value_weight, LW)
    svb = sq(src_value_bias, HF)
    dvb = sq(dst_value_bias, HF)
    evw = sq(edge_value_weight, LW2)
    evb = sq(edge_value_bias, D)
    q = sq(query, D)

    fs_p, fd_p = _stage_g(feat, src_g, dst_g)
    fs = fs_p[:E]
    fd = fd_p[:E]

    logits, value_e, m = _stage_a(fs, fd, skw, dkw, skb, dkb, ekw, ekb,
                                  svw, dvw, svb, dvb, evw, evb, q)

    m16 = jnp.tile(m, (1, H)).reshape(H * H)
    lg_flat = jnp.pad(logits, ((0, EP - E), (0, 0))).reshape(EP * H)
    val_p = jnp.pad(value_e, ((0, EP - E), (0, 0)))

    parts, z_flat = _stage_b(lg_flat, val_p, dst_b, m16)
    agg, ssum = _stage_d(parts)
    attn_flat = _stage_c2(z_flat, dst_b, ssum.reshape(NP * H))

    out_feat = agg[:N]
    attn = attn_flat.reshape(EP, H)[:E]
    return out_feat, attn


# speedup vs baseline: 2.2361x; 1.1361x over previous
import functools

import jax
import jax.numpy as jnp
from jax import lax
from jax.experimental import pallas as pl
from jax.experimental.pallas import tpu as pltpu
import jax.experimental.pallas.tpu_sc as plsc

N = 5000
E = 10000
D = 64
H = 4
DH = 16
FF = 8
HF = H * FF
LW = H * FF * D
LW2 = H * FF * DH

NW = 32
EW = 320
EP = NW * EW
NP = 5120
ROW = 80
BE = 400
BN = 128
EPS = 1e-5

@functools.cache
def _mesh():
    return plsc.VectorSubcoreMesh(core_axis_name="c", subcore_axis_name="s")


def _worker_id():
    return lax.axis_index("s") * 2 + lax.axis_index("c")


def _gather_body(feat_hbm, src_hbm, dst_hbm, fs_hbm, fd_hbm,
                 sidx, didx, srows, drows, sem):
    base = _worker_id() * EW
    pltpu.sync_copy(src_hbm.at[pl.ds(base, EW)], sidx)
    pltpu.sync_copy(dst_hbm.at[pl.ds(base, EW)], didx)
    pltpu.async_copy(feat_hbm.at[sidx], srows, sem).wait()
    pltpu.async_copy(feat_hbm.at[didx], drows, sem).wait()
    pltpu.sync_copy(srows, fs_hbm.at[pl.ds(base, EW)])
    pltpu.sync_copy(drows, fd_hbm.at[pl.ds(base, EW)])


@jax.jit
def _stage_g(feat, src_p, dst_p):
    return pl.kernel(
        _gather_body,
        out_type=[jax.ShapeDtypeStruct((EP, D), jnp.float32),
                  jax.ShapeDtypeStruct((EP, D), jnp.float32)],
        mesh=_mesh(),
        scratch_types=[
            pltpu.VMEM((EW,), jnp.int32),
            pltpu.VMEM((EW,), jnp.int32),
            pltpu.VMEM((EW, D), jnp.float32),
            pltpu.VMEM((EW, D), jnp.float32),
            pltpu.SemaphoreType.DMA,
        ],
        compiler_params=pltpu.CompilerParams(use_tc_tiling_on_sc=False, needs_layout_passes=False),
    )(feat, src_p, dst_p)


def _ln16(x):
    m = jnp.mean(x, axis=-1, keepdims=True)
    c = x - m
    v = jnp.mean(c * c, axis=-1, keepdims=True)
    return c * lax.rsqrt(v + EPS)


def _ffn_path(fst, fdt, uw, vw, ub, vb, ew, eb, r1, rex, r2):
    p = uw * fst + vw * fdt
    h1 = jnp.dot(p, r1, preferred_element_type=jnp.float32) + ub + vb
    h1 = jnp.maximum(h1, 0.0)
    h1e = jnp.dot(h1, rex, preferred_element_type=jnp.float32)
    acc = jnp.dot(ew * h1e, r2, preferred_element_type=jnp.float32) + eb
    outs = [_ln16(acc[:, h * DH:(h + 1) * DH]) for h in range(H)]
    return jnp.concatenate(outs, axis=1)


def _ffn_body(fs, fd, skw, dkw, skb, dkb, ekw, ekb,
              svw, dvw, svb, dvb, evw, evb, q,
              rt, r1, rex, r2, r3,
              logits_ref, value_ref, m_ref):
    fst = jnp.dot(fs[...], rt[...], preferred_element_type=jnp.float32)
    fdt = jnp.dot(fd[...], rt[...], preferred_element_type=jnp.float32)
    key = _ffn_path(fst, fdt, skw[...], dkw[...], skb[...], dkb[...],
                    ekw[...], ekb[...], r1[...], rex[...], r2[...])
    logits = jnp.dot(key * q[...], r3[...],
                     preferred_element_type=jnp.float32)
    logits_ref[...] = logits
    val = _ffn_path(fst, fdt, svw[...], dvw[...], svb[...], dvb[...],
                    evw[...], evb[...], r1[...], rex[...], r2[...])
    value_ref[...] = val
    bmax = jnp.max(logits, axis=0, keepdims=True)
    @pl.when(pl.program_id(0) == 0)
    def _():
        m_ref[...] = bmax
    @pl.when(pl.program_id(0) > 0)
    def _():
        m_ref[...] = jnp.maximum(m_ref[...], bmax)


@jax.jit
def _stage_a(fs, fd, skw, dkw, skb, dkb, ekw, ekb,
             svw, dvw, svb, dvb, evw, evb, q):
    ar = jnp.arange
    rt = (ar(D)[:, None] == ar(LW)[None, :] % D).astype(jnp.float32)
    r1 = (ar(LW)[:, None] // D == ar(HF)[None, :]).astype(jnp.float32)
    rex = (ar(HF)[:, None] == ar(LW2)[None, :] // DH).astype(jnp.float32)
    r2 = ((ar(LW2)[:, None] // (FF * DH) == ar(D)[None, :] // DH)
          & (ar(LW2)[:, None] % DH == ar(D)[None, :] % DH)).astype(jnp.float32)
    r3 = (ar(D)[:, None] // DH == ar(H)[None, :]).astype(jnp.float32)

    grid = E // BE
    eb_spec = lambda w: pl.BlockSpec((BE, w), lambda i: (i, 0))
    cb_spec = lambda a, b: pl.BlockSpec((a, b), lambda i: (0, 0))
    return pl.pallas_call(
        _ffn_body,
        grid=(grid,),
        in_specs=[
            eb_spec(D), eb_spec(D),
            eb_spec(LW), eb_spec(LW), eb_spec(HF), eb_spec(HF),
            eb_spec(LW2), eb_spec(D),
            eb_spec(LW), eb_spec(LW), eb_spec(HF), eb_spec(HF),
            eb_spec(LW2), eb_spec(D), eb_spec(D),
            cb_spec(D, LW), cb_spec(LW, HF), cb_spec(HF, LW2),
            cb_spec(LW2, D), cb_spec(D, H),
        ],
        out_specs=[
            pl.BlockSpec((BE, H), lambda i: (i, 0)),
            pl.BlockSpec((BE, D), lambda i: (i, 0)),
            pl.BlockSpec((1, H), lambda i: (0, 0)),
        ],
        out_shape=[
            jax.ShapeDtypeStruct((E, H), jnp.float32),
            jax.ShapeDtypeStruct((E, D), jnp.float32),
            jax.ShapeDtypeStruct((1, H), jnp.float32),
        ],
    )(fs, fd, skw, dkw, skb, dkb, ekw, ekb,
      svw, dvw, svb, dvb, evw, evb, q, rt, r1, rex, r2, r3)


def _scatter_body(lg_hbm, val_hbm, dst_hbm, m16_hbm,
                  parts_hbm, z_hbm,
                  lgv, zv, dstv, valv, rowbuf, m16v, table):
    c = lax.axis_index("c")
    s = lax.axis_index("s")
    w = s * 2 + c
    base = w * EW
    nrows = NP // 16

    def zrow(idx, _):
        r = idx // (ROW // 16)
        k = idx % (ROW // 16)
        rowbuf[r, pl.ds(k * 16, 16)] = jnp.zeros((16,), jnp.float32)
        return 0
    lax.fori_loop(0, EW * (ROW // 16), zrow, 0)
    pltpu.sync_copy(rowbuf, table.at[pl.ds(s * nrows, nrows)])
    plsc.subcore_barrier()

    pltpu.sync_copy(lg_hbm.at[pl.ds(base * H, EW * H)], lgv.at[pl.ds(0, EW * H)])
    pltpu.sync_copy(dst_hbm.at[pl.ds(base, EW)], dstv)
    pltpu.sync_copy(val_hbm.at[pl.ds(base, EW)], valv)
    pltpu.sync_copy(m16_hbm, m16v)

    mv = m16v[...]

    def zcomp(j, _):
        zj = jnp.exp(lgv[pl.ds(j * 16, 16)] - mv)
        zv[pl.ds(j * 16, 16)] = zj
        return 0
    lax.fori_loop(0, EW * H // 16, zcomp, 0)

    lanes = lax.iota(jnp.int32, 16)
    lane4 = lanes & 3

    def fill(l, _):
        for h in range(H):
            zs = plsc.load_gather(
                zv, [jnp.full((16,), l * H + h, jnp.int32)])
            rowbuf[l, pl.ds(h * DH, DH)] = valv[l, pl.ds(h * DH, DH)] * zs
        zg = plsc.load_gather(zv, [l * H + lane4])
        rowbuf[l, pl.ds(D, 16)] = jnp.where(lanes < H, zg, 0.0)
        return 0
    lax.fori_loop(0, EW, fill, 0)

    pltpu.sync_copy(rowbuf, table.at[dstv], add=True)
    plsc.subcore_barrier()

    pltpu.sync_copy(table.at[pl.ds(s * nrows, nrows)],
                    parts_hbm.at[c, pl.ds(s * nrows, nrows)])
    pltpu.sync_copy(zv.at[pl.ds(0, EW * H)], z_hbm.at[pl.ds(base * H, EW * H)])


@jax.jit
def _stage_b(lg_flat, val_p, dst_p, m16):
    return pl.kernel(
        _scatter_body,
        out_type=[jax.ShapeDtypeStruct((2, NP, ROW), jnp.float32),
                  jax.ShapeDtypeStruct((EP * H,), jnp.float32)],
        mesh=_mesh(),
        scratch_types=[
            pltpu.VMEM((EW * H,), jnp.float32),
            pltpu.VMEM((EW * H + 16,), jnp.float32),
            pltpu.VMEM((EW,), jnp.int32),
            pltpu.VMEM((EW, D), jnp.float32),
            pltpu.VMEM((EW, ROW), jnp.float32),
            pltpu.VMEM((16,), jnp.float32),
            pltpu.VMEM_SHARED((NP, ROW), jnp.float32),
        ],
        compiler_params=pltpu.CompilerParams(use_tc_tiling_on_sc=False, needs_layout_passes=False),
    )(lg_flat, val_p, dst_p, m16)


def _combine_body(parts_ref, agg_ref, ssum_ref):
    p = parts_ref[0] + parts_ref[1]
    ss = p[:, D:D + H]
    ss_safe = jnp.where(ss == 0.0, 1.0, ss)
    for h in range(H):
        x = p[:, h * DH:(h + 1) * DH] / ss_safe[:, h:h + 1]
        agg_ref[:, h * DH:(h + 1) * DH] = _ln16(x)
    ssum_ref[...] = ss


@jax.jit
def _stage_d(parts):
    return pl.pallas_call(
        _combine_body,
        grid=(NP // BN,),
        in_specs=[pl.BlockSpec((2, BN, ROW), lambda i: (0, i, 0))],
        out_specs=[pl.BlockSpec((BN, D), lambda i: (i, 0)),
                   pl.BlockSpec((BN, H), lambda i: (i, 0))],
        out_shape=[jax.ShapeDtypeStruct((NP, D), jnp.float32),
                   jax.ShapeDtypeStruct((NP, H), jnp.float32)],
    )(parts)


def _attn_body(z_hbm, dst_hbm, ss_hbm, attn_hbm, ssv, zv, dstv, attnv):
    base = _worker_id() * EW
    pltpu.sync_copy(ss_hbm, ssv)
    pltpu.sync_copy(z_hbm.at[pl.ds(base * H, EW * H)], zv)
    pltpu.sync_copy(dst_hbm.at[pl.ds(base, EW)], dstv)

    lanes = lax.iota(jnp.int32, 16)
    lsh = lanes >> 2
    lane4 = lanes & 3

    def step(j, _):
        dg = plsc.load_gather(dstv, [j * 4 + lsh])
        sv = plsc.load_gather(ssv, [dg * H + lane4])
        attnv[pl.ds(j * 16, 16)] = zv[pl.ds(j * 16, 16)] / sv
        return 0
    lax.fori_loop(0, EW * H // 16, step, 0)
    pltpu.sync_copy(attnv, attn_hbm.at[pl.ds(base * H, EW * H)])


@jax.jit
def _stage_c2(z_flat, dst_p, ss_flat):
    return pl.kernel(
        _attn_body,
        out_type=jax.ShapeDtypeStruct((EP * H,), jnp.float32),
        mesh=_mesh(),
        scratch_types=[
            pltpu.VMEM((NP * H,), jnp.float32),
            pltpu.VMEM((EW * H,), jnp.float32),
            pltpu.VMEM((EW,), jnp.int32),
            pltpu.VMEM((EW * H,), jnp.float32),
        ],
        compiler_params=pltpu.CompilerParams(use_tc_tiling_on_sc=False, needs_layout_passes=False),
    )(z_flat, dst_p, ss_flat)


def kernel(feat, edge_index, query, src_key_weight, dst_key_weight,
           src_key_bias, dst_key_bias, edge_key_weight, edge_key_bias,
           src_value_weight, dst_value_weight, src_value_bias, dst_value_bias,
           edge_value_weight, edge_value_bias, key_norm_gamma, key_norm_beta,
           value_norm_gamma, value_norm_beta, out_norm_gamma, out_norm_beta, i):
    f32 = jnp.float32
    ei = edge_index.astype(jnp.int32)
    src = ei[0]
    dst = ei[1]
    src_g = jnp.pad(src, (0, EP - E))
    dst_g = jnp.pad(dst, (0, EP - E))
    dst_b = jnp.pad(dst, (0, EP - E), constant_values=N)

    sq = lambda w, s: w.reshape(E, s).astype(f32)
    skw = sq(src_key_weight, LW)
    dkw = sq(dst_key_weight, LW)
    skb = sq(src_key_bias, HF)
    dkb = sq(dst_key_bias, HF)
    ekw = sq(edge_key_weight, LW2)
    ekb = sq(edge_key_bias, D)
    svw = sq(src_value_weight, LW)
    dvw = sq(dst_value_weight, LW)
    svb = sq(src_value_bias, HF)
    dvb = sq(dst_value_bias, HF)
    evw = sq(edge_value_weight, LW2)
    evb = sq(edge_value_bias, D)
    q = sq(query, D)

    fs_p, fd_p = _stage_g(feat, src_g, dst_g)
    fs = fs_p[:E]
    fd = fd_p[:E]

    logits, value_e, m = _stage_a(fs, fd, skw, dkw, skb, dkb, ekw, ekb,
                                  svw, dvw, svb, dvb, evw, evb, q)

    m16 = jnp.tile(m, (1, H)).reshape(H * H)
    lg_flat = jnp.pad(logits, ((0, EP - E), (0, 0))).reshape(EP * H)
    val_p = jnp.pad(value_e, ((0, EP - E), (0, 0)))

    parts, z_flat = _stage_b(lg_flat, val_p, dst_b, m16)
    agg, ssum = _stage_d(parts)
    attn_flat = _stage_c2(z_flat, dst_b, ssum.reshape(NP * H))

    out_feat = agg[:N]
    attn = attn_flat.reshape(EP, H)[:E]
    return out_feat, attn
